# baseline (speedup 1.0000x reference)
import functools

import jax
import jax.numpy as jnp
from jax import lax
from jax.experimental import pallas as pl
from jax.experimental.pallas import tpu as pltpu
from jax.experimental.pallas import tpu_sc as plsc

B = 64
L = 2048
D = 64
NC = 2
NS = 16
NW = NC * NS
BPW = B // NW
C = 256
G = 256
NG = C // G
NCHUNK = L // C
DP = 2 * D


CW = 512
V = 1000000


def _tp_body(tabT_ref, o_ref):
    o_ref[:, :D] = tabT_ref[...].T


def _transpose_table(tabT):
    return pl.pallas_call(
        _tp_body,
        out_shape=jax.ShapeDtypeStruct((V, 2 * D), jnp.float32),
        grid=(pl.cdiv(V, CW),),
        in_specs=[pl.BlockSpec((D, CW), lambda i: (0, i))],
        out_specs=pl.BlockSpec((CW, 2 * D), lambda i: (i, 0)),
    )(tabT)


def _body(x_hbm, tab_hbm, pos_hbm, out_hbm, idx_v, pos_v, rows_v, out_v, sem):
    wid = lax.axis_index("s") * NC + lax.axis_index("c")

    def chunk_step(c, carry):
        pltpu.sync_copy(pos_hbm.at[pl.ds(c * C, C), :], pos_v)
        for b in range(BPW):
            row0 = (wid * BPW + b) * L + c * C
            pltpu.sync_copy(x_hbm.at[pl.ds(row0, C)], idx_v)
            copies = [
                pltpu.async_copy(
                    tab_hbm.at[idx_v.at[pl.ds(j * G, G)]],
                    rows_v.at[pl.ds(j * G, G), :],
                    sem,
                )
                for j in range(NG)
            ]
            for cp in copies:
                cp.wait()

            def add_step(i, carry2):
                for u in range(2):
                    r = i * 2 + u
                    for d in range(D // 16):
                        out_v[r, pl.ds(d * 16, 16)] = (
                            rows_v[r, pl.ds(d * 16, 16)]
                            + pos_v[r, pl.ds(d * 16, 16)]
                        )
                return carry2

            lax.fori_loop(0, C // 2, add_step, 0)
            pltpu.sync_copy(out_v, out_hbm.at[pl.ds(row0, C), :])
        return carry

    lax.fori_loop(0, NCHUNK, chunk_step, 0)


@jax.jit
def kernel(x, token_embedding, pos_encoding):
    x1d = x.astype(jnp.int32).reshape(B * L)
    tab_p = _transpose_table(token_embedding.T)
    mesh = plsc.VectorSubcoreMesh(core_axis_name="c", subcore_axis_name="s")
    out = pl.kernel(
        _body,
        out_type=jax.ShapeDtypeStruct((B * L, D), jnp.float32),
        mesh=mesh,
        scratch_types=[
            pltpu.VMEM((C,), jnp.int32),
            pltpu.VMEM((C, D), jnp.float32),
            pltpu.VMEM((C, DP), jnp.float32),
            pltpu.VMEM((C, D), jnp.float32),
            pltpu.SemaphoreType.DMA,
        ],
        compiler_params=pltpu.CompilerParams(use_tc_tiling_on_sc=False),
    )(x1d, tab_p, pos_encoding)
    return out.reshape(B, L, D)


# speedup vs baseline: 1.9079x; 1.9079x over previous
import functools

import jax
import jax.numpy as jnp
from jax import lax
from jax.experimental import pallas as pl
from jax.experimental.pallas import tpu as pltpu
from jax.experimental.pallas import tpu_sc as plsc

B = 64
L = 2048
D = 64
NC = 2
NS = 16
NW = NC * NS
BPW = B // NW
C = 512
NCHUNK = L // C


def _body(x_hbm, tab_hbm, pos_hbm, out_hbm, idx_v, pos_v, rows_v, sem):
    wid = lax.axis_index("s") * NC + lax.axis_index("c")

    def chunk_step(c, carry):
        pltpu.sync_copy(pos_hbm.at[pl.ds(c * C, C), :], pos_v)
        for b in range(BPW):
            row0 = (wid * BPW + b) * L + c * C
            pltpu.sync_copy(x_hbm.at[pl.ds(row0, C)], idx_v)
            pltpu.async_copy(tab_hbm.at[idx_v], rows_v, sem).wait()

            def add_step(i, carry2):
                for u in range(2):
                    r = i * 2 + u
                    for d in range(D // 16):
                        plsc.addupdate(
                            rows_v.at[r, pl.ds(d * 16, 16)],
                            pos_v[r, pl.ds(d * 16, 16)],
                        )
                return carry2

            lax.fori_loop(0, C // 2, add_step, 0)
            pltpu.sync_copy(rows_v, out_hbm.at[pl.ds(row0, C), :])
        return carry

    lax.fori_loop(0, NCHUNK, chunk_step, 0)


@jax.jit
def kernel(x, token_embedding, pos_encoding):
    x1d = x.astype(jnp.int32).reshape(B * L)
    mesh = plsc.VectorSubcoreMesh(core_axis_name="c", subcore_axis_name="s")
    out = pl.kernel(
        _body,
        out_type=jax.ShapeDtypeStruct((B * L, D), jnp.float32),
        mesh=mesh,
        scratch_types=[
            pltpu.VMEM((C,), jnp.int32),
            pltpu.VMEM((C, D), jnp.float32),
            pltpu.VMEM((C, D), jnp.float32),
            pltpu.SemaphoreType.DMA,
        ],
        compiler_params=pltpu.CompilerParams(use_tc_tiling_on_sc=False),
    )(x1d, token_embedding, pos_encoding)
    return out.reshape(B, L, D)
